# baseline (speedup 1.0000x reference)
import functools

import jax
import jax.numpy as jnp
from jax import lax
from jax.experimental import pallas as pl
from jax.experimental.pallas import tpu as pltpu
from jax.experimental.pallas import tpu_sc as plsc

B, D2, V, D = 16384, 64, 100000, 128
NC, NS = 2, 16
NW = NC * NS
BPW = B // NW
NCH = 4
CH = BPW // NCH

BB = 2048


@functools.cache
def _make_sc_gather():
    mesh = plsc.VectorSubcoreMesh(core_axis_name="c", subcore_axis_name="s")

    @functools.partial(
        pl.kernel,
        mesh=mesh,
        out_type=jax.ShapeDtypeStruct((NW, NCH, CH, D), jnp.float32),
        scratch_types=[
            pltpu.VMEM((NCH, CH), jnp.int32),
            pltpu.VMEM((NCH, CH, D), jnp.float32),
            pltpu.SemaphoreType.DMA,
        ],
    )
    def sc_gather(table_hbm, idx_hbm, out_hbm, idx_v, rows_v, sem):
        wid = lax.axis_index("s") * NC + lax.axis_index("c")
        pltpu.sync_copy(idx_hbm.at[wid], idx_v)
        descs = [
            pltpu.async_copy(table_hbm.at[idx_v.at[c]], rows_v.at[c], sem)
            for c in range(NCH)
        ]
        for d in descs:
            d.wait()
        pltpu.sync_copy(rows_v, out_hbm.at[wid])

    return sc_gather


def _tc_body(lam_ref, lam2_ref, emb_ref, hhm_ref, htm_ref, hhp_ref, htp_ref,
             out_ref):
    x = emb_ref[:, :D2]
    y = emb_ref[:, D2:]
    r = jnp.sqrt(x * x + y * y)

    t = hhm_ref[:] * r - htm_ref[:]
    d_m = jnp.sqrt(jnp.sum(t * t, axis=1, keepdims=True))

    a = (hhp_ref[:] - htp_ref[:]) * 0.5
    sa = jnp.sin(a)
    ca = jnp.cos(a)
    sgn = jnp.where(y >= 0.0, 1.0, -1.0)
    num = (ca * jnp.sqrt(jnp.maximum(r + x, 0.0))
           - sgn * sa * jnp.sqrt(jnp.maximum(r - x, 0.0)))
    inv = lax.rsqrt(jnp.maximum(2.0 * r, 1e-30))
    d_p = jnp.sum(jnp.abs(num) * inv, axis=1, keepdims=True)

    out_ref[:] = -(lam2_ref[0] * d_m + lam_ref[0] * d_p)


def kernel(h_head_m, h_tail_m, h_head_p, h_tail_p, rels, W, lam, lam2):
    idx = rels.astype(jnp.int32).reshape(NW, NCH, CH)
    emb = _make_sc_gather()(W, idx).reshape(B, D)

    score = pl.pallas_call(
        _tc_body,
        grid=(B // BB,),
        in_specs=[
            pl.BlockSpec(memory_space=pltpu.SMEM),
            pl.BlockSpec(memory_space=pltpu.SMEM),
            pl.BlockSpec((BB, D), lambda i: (i, 0)),
            pl.BlockSpec((BB, D2), lambda i: (i, 0)),
            pl.BlockSpec((BB, D2), lambda i: (i, 0)),
            pl.BlockSpec((BB, D2), lambda i: (i, 0)),
            pl.BlockSpec((BB, D2), lambda i: (i, 0)),
        ],
        out_specs=pl.BlockSpec((BB, 1), lambda i: (i, 0)),
        out_shape=jax.ShapeDtypeStruct((B, 1), jnp.float32),
    )(lam, lam2, emb, h_head_m, h_tail_m, h_head_p, h_tail_p)
    return score.reshape(B)
